# baseline (speedup 1.0000x reference)
import functools

import jax
import jax.numpy as jnp
from jax.experimental import pallas as pl
from jax.experimental.pallas import tpu as pltpu

_BM = 400
_CACHE = 2


def _fused_kernel(adj_ref, x_ref, w1_ref, b1_ref, w2_ref, b2_ref, o_ref,
                  h1w_s, cache_s, *, nb, bm, nc):
    i = pl.program_id(0)

    @pl.when(i < nb)
    def _():
        s = jnp.dot(adj_ref[...], x_ref[...], preferred_element_type=jnp.float32)
        h = jnp.dot(
            s, w1_ref[...], preferred_element_type=jnp.float32,
        ) + b1_ref[...]
        h = jnp.where(h >= 0, h, 0.01 * h)
        h1w_s[pl.ds(i * bm, bm), :] = jnp.dot(
            h, w2_ref[...], preferred_element_type=jnp.float32,
        ).astype(jnp.bfloat16)

    @pl.when((i >= nb - 1 - nc) & (i <= nb - 2))
    def _():
        cache_s[pl.ds((i - (nb - 1 - nc)) * bm, bm), :] = (
            adj_ref[...].astype(jnp.bfloat16))

    @pl.when((i == nb) | (i > nb + nc))
    def _():
        acc = jnp.dot(adj_ref[...], h1w_s[...], preferred_element_type=jnp.float32)
        h = acc + b2_ref[...]
        o_ref[...] = jnp.where(h >= 0, h, 0.01 * h)

    @pl.when((i > nb) & (i <= nb + nc))
    def _():
        blk = cache_s[pl.ds((nc - (i - nb)) * bm, bm), :]
        acc = jnp.dot(blk, h1w_s[...], preferred_element_type=jnp.float32)
        h = acc + b2_ref[...]
        o_ref[...] = jnp.where(h >= 0, h, 0.01 * h)


def kernel(x, adj, W1, b1, W2, b2):
    n, d = adj.shape[0], W1.shape[1]
    nb = n // _BM
    nc = _CACHE
    b1r = b1.reshape(1, -1)
    b2r = b2.reshape(1, -1)
    xb = x.astype(jnp.bfloat16)

    def _adj_idx(i):
        return (
            jnp.where(
                i < nb, i,
                jnp.where(i <= nb + nc, nb - 1, i - nb - 1 - nc),
            ),
            0,
        )

    def _out_idx(i):
        return (
            jnp.where(
                i <= nb, nb - 1,
                jnp.where(i <= nb + nc, nb - 1 - (i - nb), i - nb - 1 - nc),
            ),
            0,
        )

    return pl.pallas_call(
        functools.partial(_fused_kernel, nb=nb, bm=_BM, nc=nc),
        grid=(2 * nb,),
        in_specs=[
            pl.BlockSpec((_BM, n), _adj_idx),
            pl.BlockSpec(xb.shape, lambda i: (0, 0)),
            pl.BlockSpec(W1.shape, lambda i: (0, 0)),
            pl.BlockSpec(b1r.shape, lambda i: (0, 0)),
            pl.BlockSpec(W2.shape, lambda i: (0, 0)),
            pl.BlockSpec(b2r.shape, lambda i: (0, 0)),
        ],
        out_specs=pl.BlockSpec((_BM, d), _out_idx),
        out_shape=jax.ShapeDtypeStruct((n, d), jnp.float32),
        scratch_shapes=[
            pltpu.VMEM((n, d), jnp.bfloat16),
            pltpu.VMEM((nc * _BM, n), jnp.bfloat16),
        ],
    )(adj, xb, W1, b1r, W2, b2r)


# speedup vs baseline: 1.0034x; 1.0034x over previous
import functools

import jax
import jax.numpy as jnp
from jax.experimental import pallas as pl
from jax.experimental.pallas import tpu as pltpu

_BM = 400
_CACHE = 2


def _fused_kernel(adj_ref, x_ref, w1_ref, b1_ref, w2_ref, b2_ref, o_ref,
                  h1w_s, cache_s, *, nb, bm, nc):
    i = pl.program_id(0)

    @pl.when(i < nb)
    def _():
        s = jnp.dot(adj_ref[...], x_ref[...], preferred_element_type=jnp.float32)
        h = jnp.dot(
            s, w1_ref[...], preferred_element_type=jnp.float32,
        ) + b1_ref[...]
        h = jnp.where(h >= 0, h, 0.01 * h)
        h1w_s[pl.ds(i * bm, bm), :] = jnp.dot(
            h, w2_ref[...], preferred_element_type=jnp.float32,
        )

    @pl.when((i >= nb - 1 - nc) & (i <= nb - 2))
    def _():
        cache_s[pl.ds((i - (nb - 1 - nc)) * bm, bm), :] = (
            adj_ref[...].astype(jnp.bfloat16))

    @pl.when((i == nb) | (i > nb + nc))
    def _():
        acc = jnp.dot(adj_ref[...], h1w_s[...], preferred_element_type=jnp.float32)
        h = acc + b2_ref[...]
        o_ref[...] = jnp.where(h >= 0, h, 0.01 * h)

    @pl.when((i > nb) & (i <= nb + nc))
    def _():
        blk = cache_s[pl.ds((nc - (i - nb)) * bm, bm), :]
        acc = jnp.dot(blk, h1w_s[...], preferred_element_type=jnp.float32)
        h = acc + b2_ref[...]
        o_ref[...] = jnp.where(h >= 0, h, 0.01 * h)


def kernel(x, adj, W1, b1, W2, b2):
    n, d = adj.shape[0], W1.shape[1]
    nb = n // _BM
    nc = _CACHE
    b1r = b1.reshape(1, -1)
    b2r = b2.reshape(1, -1)
    xb = x.astype(jnp.bfloat16)

    def _adj_idx(i):
        return (
            jnp.where(
                i < nb, i,
                jnp.where(i <= nb + nc, nb - 1, i - nb - 1 - nc),
            ),
            0,
        )

    def _out_idx(i):
        return (
            jnp.where(
                i <= nb, nb - 1,
                jnp.where(i <= nb + nc, nb - 1 - (i - nb), i - nb - 1 - nc),
            ),
            0,
        )

    return pl.pallas_call(
        functools.partial(_fused_kernel, nb=nb, bm=_BM, nc=nc),
        grid=(2 * nb,),
        in_specs=[
            pl.BlockSpec((_BM, n), _adj_idx),
            pl.BlockSpec(xb.shape, lambda i: (0, 0)),
            pl.BlockSpec(W1.shape, lambda i: (0, 0)),
            pl.BlockSpec(b1r.shape, lambda i: (0, 0)),
            pl.BlockSpec(W2.shape, lambda i: (0, 0)),
            pl.BlockSpec(b2r.shape, lambda i: (0, 0)),
        ],
        out_specs=pl.BlockSpec((_BM, d), _out_idx),
        out_shape=jax.ShapeDtypeStruct((n, d), jnp.float32),
        scratch_shapes=[
            pltpu.VMEM((n, d), jnp.float32),
            pltpu.VMEM((nc * _BM, n), jnp.bfloat16),
        ],
    )(adj, xb, W1, b1r, W2, b2r)


# speedup vs baseline: 1.0169x; 1.0134x over previous
import functools

import jax
import jax.numpy as jnp
from jax.experimental import pallas as pl
from jax.experimental.pallas import tpu as pltpu

_BM = 400
_CACHE = 2


def _fused_kernel(adj_ref, x_ref, w1_ref, b1_ref, w2_ref, b2_ref, o_ref,
                  h1w_s, cache_s, *, nb, bm, nc):
    i = pl.program_id(0)

    @pl.when(i < nb)
    def _():
        s = jnp.dot(adj_ref[...], x_ref[...], preferred_element_type=jnp.float32)
        h = jnp.dot(
            s, w1_ref[...], preferred_element_type=jnp.float32,
        ) + b1_ref[...]
        h = jnp.where(h >= 0, h, 0.01 * h)
        h1w_s[pl.ds(i * bm, bm), :] = jnp.dot(
            h, w2_ref[...], preferred_element_type=jnp.float32,
        )

    @pl.when((i >= nb - 1 - nc) & (i <= nb - 2))
    def _():
        cache_s[pl.ds((i - (nb - 1 - nc)) * bm, bm), :] = (
            adj_ref[...].astype(jnp.bfloat16))

    @pl.when((i == nb) | (i > nb + nc))
    def _():
        acc = jnp.dot(adj_ref[...], h1w_s[...], preferred_element_type=jnp.float32)
        h = acc + b2_ref[...]
        o_ref[...] = jnp.where(h >= 0, h, 0.01 * h)

    @pl.when((i > nb) & (i <= nb + nc))
    def _():
        blk = cache_s[pl.ds((nc - (i - nb)) * bm, bm), :]
        acc = jnp.dot(blk, h1w_s[...], preferred_element_type=jnp.float32)
        h = acc + b2_ref[...]
        o_ref[...] = jnp.where(h >= 0, h, 0.01 * h)


def kernel(x, adj, W1, b1, W2, b2):
    n, d = adj.shape[0], W1.shape[1]
    nb = n // _BM
    nc = _CACHE
    b1r = b1.reshape(1, -1)
    b2r = b2.reshape(1, -1)
    xb = x

    def _adj_idx(i):
        return (
            jnp.where(
                i < nb, i,
                jnp.where(i <= nb + nc, nb - 1, i - nb - 1 - nc),
            ),
            0,
        )

    def _out_idx(i):
        return (
            jnp.where(
                i <= nb, nb - 1,
                jnp.where(i <= nb + nc, nb - 1 - (i - nb), i - nb - 1 - nc),
            ),
            0,
        )

    return pl.pallas_call(
        functools.partial(_fused_kernel, nb=nb, bm=_BM, nc=nc),
        grid=(2 * nb,),
        in_specs=[
            pl.BlockSpec((_BM, n), _adj_idx),
            pl.BlockSpec(xb.shape, lambda i: (0, 0)),
            pl.BlockSpec(W1.shape, lambda i: (0, 0)),
            pl.BlockSpec(b1r.shape, lambda i: (0, 0)),
            pl.BlockSpec(W2.shape, lambda i: (0, 0)),
            pl.BlockSpec(b2r.shape, lambda i: (0, 0)),
        ],
        out_specs=pl.BlockSpec((_BM, d), _out_idx),
        out_shape=jax.ShapeDtypeStruct((n, d), jnp.float32),
        scratch_shapes=[
            pltpu.VMEM((n, d), jnp.float32),
            pltpu.VMEM((nc * _BM, n), jnp.bfloat16),
        ],
    )(adj, xb, W1, b1r, W2, b2r)
